# baseline (device time: 6698 ns/iter reference)
import os

import jax
import jax.numpy as jnp
from jax import lax
from jax.experimental import pallas as pl
from jax.experimental.pallas import tpu as pltpu

X, Y, Z = 2, 2, 4
EPS = 1e-5
SPLIT = int(os.environ.get("LNBWD_SPLIT", "4"))
_SKIP_COMM = os.environ.get("LNBWD_SKIP_COMM", "0") == "1"


def kernel(x, dy, gamma):
    m, d = x.shape
    mq = m // SPLIT
    G = SPLIT * Z

    def body(x_ref, dy_ref, gamma_ref, out_ref,
             xv_ref, dyv_ref, comm_ref, copy_sems, send_sems, recv_sems):
        my_x = lax.axis_index("x")
        my_y = lax.axis_index("y")
        my_z = lax.axis_index("z")
        if SPLIT == 4:
            q = my_x * Y + my_y
            me = q * Z + my_z
            member = lambda t: (t // (Y * Z), (t // Z) % Y, t % Z)
        elif SPLIT == 2:
            q = my_x
            me = my_x * Z + my_z
            member = lambda t: (t // Z, my_y, t % Z)
        else:
            q = my_x * 0
            me = my_z
            member = lambda t: (my_x, my_y, t)

        barrier_sem = None
        if not _SKIP_COMM:
            barrier_sem = pltpu.get_barrier_semaphore()
            for t in range(G):
                @pl.when(t != me)
                def _(t=t):
                    pl.semaphore_signal(
                        barrier_sem, inc=1,
                        device_id=member(t),
                        device_id_type=pl.DeviceIdType.MESH,
                    )

        cp_x = pltpu.make_async_copy(
            x_ref.at[pl.ds(q * mq, mq), :], xv_ref, copy_sems.at[0])
        cp_dy = pltpu.make_async_copy(
            dy_ref.at[pl.ds(q * mq, mq), :], dyv_ref, copy_sems.at[1])
        cp_x.start()
        cp_dy.start()
        cp_x.wait()
        cp_dy.wait()

        xv = xv_ref[...]
        dyv = dyv_ref[...]
        mu = jnp.mean(xv, axis=1, keepdims=True)
        xc = xv - mu
        var = jnp.mean(xc * xc, axis=1, keepdims=True)
        xhat = xc * lax.rsqrt(var + EPS)
        dgamma = jnp.sum(dyv * xhat, axis=0, keepdims=True)
        dbeta = jnp.sum(dyv, axis=0, keepdims=True)
        comm_ref[pl.ds(me, 1)] = jnp.concatenate([dgamma, dbeta], axis=0)[None]

        if _SKIP_COMM:
            out_ref[...] = jnp.sum(comm_ref[...], axis=0)
            return

        pl.semaphore_wait(barrier_sem, G - 1)

        sends = []
        for t in range(G):
            rdma = pltpu.make_async_remote_copy(
                src_ref=comm_ref.at[me],
                dst_ref=comm_ref.at[me],
                send_sem=send_sems.at[t],
                recv_sem=recv_sems.at[me],
                device_id=member(t),
                device_id_type=pl.DeviceIdType.MESH,
            )
            @pl.when(t != me)
            def _(rdma=rdma):
                rdma.start()
            sends.append(rdma)

        for s in range(G):
            recv = pltpu.make_async_remote_copy(
                src_ref=comm_ref.at[s],
                dst_ref=comm_ref.at[s],
                send_sem=send_sems.at[s],
                recv_sem=recv_sems.at[s],
                device_id=member(s),
                device_id_type=pl.DeviceIdType.MESH,
            )
            @pl.when(s != me)
            def _(recv=recv):
                recv.wait_recv()

        out_ref[...] = jnp.sum(comm_ref[...], axis=0)

        for t, rdma in enumerate(sends):
            @pl.when(t != me)
            def _(rdma=rdma):
                rdma.wait_send()

    return pl.pallas_call(
        body,
        out_shape=jax.ShapeDtypeStruct((2, d), jnp.float32),
        in_specs=[
            pl.BlockSpec(memory_space=pl.ANY),
            pl.BlockSpec(memory_space=pl.ANY),
            pl.BlockSpec(memory_space=pl.ANY),
        ],
        out_specs=pl.BlockSpec(memory_space=pltpu.VMEM),
        scratch_shapes=[
            pltpu.VMEM((mq, d), jnp.float32),
            pltpu.VMEM((mq, d), jnp.float32),
            pltpu.VMEM((G, 2, d), jnp.float32),
            pltpu.SemaphoreType.DMA((2,)),
            pltpu.SemaphoreType.DMA((G,)),
            pltpu.SemaphoreType.DMA((G,)),
        ],
        compiler_params=(
            None if _SKIP_COMM else pltpu.CompilerParams(collective_id=0)
        ),
    )(x, dy, gamma)
